# baseline (speedup 1.0000x reference)
import jax
import jax.numpy as jnp
from jax.experimental import pallas as pl
from jax.experimental.pallas import tpu as pltpu


def _fused_kernel(x_ref, w_ref, m_ref, b_ref, o_ref, wbf_ref):
    @pl.when(pl.program_id(1) == 0)
    def _():
        wbf_ref[...] = (w_ref[...] * m_ref[...]).astype(jnp.bfloat16)

    y = jnp.dot(x_ref[...].astype(jnp.bfloat16), wbf_ref[...],
                preferred_element_type=jnp.float32)
    o_ref[...] = jnp.maximum(y + b_ref[...], 0.0).astype(o_ref.dtype)


def kernel(x, weight, mask, bias):
    B, n_in = x.shape
    n_out = weight.shape[1]
    bias2d = bias.astype(jnp.float32).reshape(1, n_out)

    tm = 256
    n_cores = 2
    steps = B // tm // n_cores

    out = pl.pallas_call(
        _fused_kernel,
        out_shape=jax.ShapeDtypeStruct((B, n_out), x.dtype),
        grid=(n_cores, steps),
        in_specs=[
            pl.BlockSpec((tm, n_in), lambda i, j: (i * steps + j, 0)),
            pl.BlockSpec((n_in, n_out), lambda i, j: (0, 0)),
            pl.BlockSpec((n_in, n_out), lambda i, j: (0, 0)),
            pl.BlockSpec((1, n_out), lambda i, j: (0, 0)),
        ],
        out_specs=pl.BlockSpec((tm, n_out), lambda i, j: (i * steps + j, 0)),
        scratch_shapes=[pltpu.VMEM((n_in, n_out), jnp.bfloat16)],
        compiler_params=pltpu.CompilerParams(
            dimension_semantics=("parallel", "arbitrary")),
    )(x, weight, mask, bias2d)
    return out


# speedup vs baseline: 1.0044x; 1.0044x over previous
import jax
import jax.numpy as jnp
from jax.experimental import pallas as pl
from jax.experimental.pallas import tpu as pltpu


def _mask_fold_kernel(w_ref, m_ref, o_ref):
    o_ref[...] = (w_ref[...] * m_ref[...]).astype(jnp.bfloat16)


def _rows_kernel(x_ref, w_ref, b_ref, o_ref):
    y = jnp.dot(x_ref[...].astype(jnp.bfloat16), w_ref[...],
                preferred_element_type=jnp.float32)
    o_ref[...] = jnp.maximum(y + b_ref[...], 0.0).astype(o_ref.dtype)


def kernel(x, weight, mask, bias):
    B, n_in = x.shape
    n_out = weight.shape[1]

    fold_grid = 8
    fold_rows = n_in // fold_grid
    w_bf16 = pl.pallas_call(
        _mask_fold_kernel,
        out_shape=jax.ShapeDtypeStruct((n_in, n_out), jnp.bfloat16),
        grid=(fold_grid,),
        in_specs=[
            pl.BlockSpec((fold_rows, n_out), lambda i: (i, 0)),
            pl.BlockSpec((fold_rows, n_out), lambda i: (i, 0)),
        ],
        out_specs=pl.BlockSpec((fold_rows, n_out), lambda i: (i, 0)),
        compiler_params=pltpu.CompilerParams(
            dimension_semantics=("parallel",)),
    )(weight, mask)

    bias2d = bias.astype(jnp.float32).reshape(1, n_out)

    tm = 1024
    out = pl.pallas_call(
        _rows_kernel,
        out_shape=jax.ShapeDtypeStruct((B, n_out), x.dtype),
        grid=(B // tm,),
        in_specs=[
            pl.BlockSpec((tm, n_in), lambda i: (i, 0)),
            pl.BlockSpec((n_in, n_out), lambda i: (0, 0)),
            pl.BlockSpec((1, n_out), lambda i: (0, 0)),
        ],
        out_specs=pl.BlockSpec((tm, n_out), lambda i: (i, 0)),
        compiler_params=pltpu.CompilerParams(
            dimension_semantics=("parallel",)),
    )(x, w_bf16, bias2d)
    return out


# speedup vs baseline: 1.0109x; 1.0065x over previous
import jax
import jax.numpy as jnp
from jax.experimental import pallas as pl
from jax.experimental.pallas import tpu as pltpu


def _mask_fold_kernel(w_ref, m_ref, o_ref):
    o_ref[...] = (w_ref[...] * m_ref[...]).astype(jnp.bfloat16)


def _rows_kernel(x_ref, w_ref, b_ref, o_ref):
    y = jnp.dot(x_ref[...].astype(jnp.bfloat16), w_ref[...],
                preferred_element_type=jnp.float32)
    o_ref[...] = jnp.maximum(y + b_ref[...], 0.0).astype(o_ref.dtype)


def kernel(x, weight, mask, bias):
    B, n_in = x.shape
    n_out = weight.shape[1]

    fold_grid = 8
    fold_rows = n_in // fold_grid
    w_bf16 = pl.pallas_call(
        _mask_fold_kernel,
        out_shape=jax.ShapeDtypeStruct((n_in, n_out), jnp.bfloat16),
        grid=(fold_grid,),
        in_specs=[
            pl.BlockSpec((fold_rows, n_out), lambda i: (i, 0)),
            pl.BlockSpec((fold_rows, n_out), lambda i: (i, 0)),
        ],
        out_specs=pl.BlockSpec((fold_rows, n_out), lambda i: (i, 0)),
        compiler_params=pltpu.CompilerParams(
            dimension_semantics=("parallel",)),
    )(weight, mask)

    bias2d = bias.astype(jnp.float32).reshape(1, n_out)

    tm = 512
    out = pl.pallas_call(
        _rows_kernel,
        out_shape=jax.ShapeDtypeStruct((B, n_out), x.dtype),
        grid=(B // tm,),
        in_specs=[
            pl.BlockSpec((tm, n_in), lambda i: (i, 0)),
            pl.BlockSpec((n_in, n_out), lambda i: (0, 0)),
            pl.BlockSpec((1, n_out), lambda i: (0, 0)),
        ],
        out_specs=pl.BlockSpec((tm, n_out), lambda i: (i, 0)),
        compiler_params=pltpu.CompilerParams(
            dimension_semantics=("parallel",)),
    )(x, w_bf16, bias2d)
    return out


# speedup vs baseline: 1.0295x; 1.0183x over previous
import jax
import jax.numpy as jnp
from jax.experimental import pallas as pl
from jax.experimental.pallas import tpu as pltpu


def _fused_kernel(x_ref, w_ref, m_ref, b_ref, o_ref, wbf_ref):
    @pl.when(pl.program_id(0) == 0)
    def _():
        wbf_ref[...] = (w_ref[...] * m_ref[...]).astype(jnp.bfloat16)

    y = jnp.dot(x_ref[...].astype(jnp.bfloat16), wbf_ref[...],
                preferred_element_type=jnp.float32)
    o_ref[...] = jnp.maximum(y + b_ref[...], 0.0).astype(o_ref.dtype)


def kernel(x, weight, mask, bias):
    B, n_in = x.shape
    n_out = weight.shape[1]
    bias2d = bias.astype(jnp.float32).reshape(1, n_out)

    tm = 256
    out = pl.pallas_call(
        _fused_kernel,
        out_shape=jax.ShapeDtypeStruct((B, n_out), x.dtype),
        grid=(B // tm,),
        in_specs=[
            pl.BlockSpec((tm, n_in), lambda i: (i, 0)),
            pl.BlockSpec((n_in, n_out), lambda i: (0, 0)),
            pl.BlockSpec((n_in, n_out), lambda i: (0, 0)),
            pl.BlockSpec((1, n_out), lambda i: (0, 0)),
        ],
        out_specs=pl.BlockSpec((tm, n_out), lambda i: (i, 0)),
        scratch_shapes=[pltpu.VMEM((n_in, n_out), jnp.bfloat16)],
        compiler_params=pltpu.CompilerParams(
            dimension_semantics=("arbitrary",)),
    )(x, weight, mask, bias2d)
    return out


# speedup vs baseline: 1.0792x; 1.0483x over previous
import jax
import jax.numpy as jnp
from jax.experimental import pallas as pl
from jax.experimental.pallas import tpu as pltpu


def _fused_kernel(x_ref, w_ref, m_ref, b_ref, o_ref, wbf_ref):
    @pl.when(pl.program_id(0) == 0)
    def _():
        wbf_ref[...] = (w_ref[...] * m_ref[...]).astype(jnp.bfloat16)

    y = jnp.dot(x_ref[...].astype(jnp.bfloat16), wbf_ref[...],
                preferred_element_type=jnp.float32)
    o_ref[...] = jnp.maximum(y + b_ref[...], 0.0).astype(o_ref.dtype)


def kernel(x, weight, mask, bias):
    B, n_in = x.shape
    n_out = weight.shape[1]
    bias2d = bias.astype(jnp.float32).reshape(1, n_out)

    tm = 512
    out = pl.pallas_call(
        _fused_kernel,
        out_shape=jax.ShapeDtypeStruct((B, n_out), x.dtype),
        grid=(B // tm,),
        in_specs=[
            pl.BlockSpec((tm, n_in), lambda i: (i, 0)),
            pl.BlockSpec((n_in, n_out), lambda i: (0, 0)),
            pl.BlockSpec((n_in, n_out), lambda i: (0, 0)),
            pl.BlockSpec((1, n_out), lambda i: (0, 0)),
        ],
        out_specs=pl.BlockSpec((tm, n_out), lambda i: (i, 0)),
        scratch_shapes=[pltpu.VMEM((n_in, n_out), jnp.bfloat16)],
        compiler_params=pltpu.CompilerParams(
            dimension_semantics=("arbitrary",)),
    )(x, weight, mask, bias2d)
    return out
